# Initial kernel scaffold; baseline (speedup 1.0000x reference)
import jax
import jax.numpy as jnp
from jax.experimental import pallas as pl


def kernel(x, node_idx, he_idx, y, batch_0, W0, b0, W1, b1):
    raise NotImplementedError("write your pallas kernel here")


# speedup vs baseline: 6.7506x; 6.7506x over previous
import functools

import jax
import jax.numpy as jnp
from jax import lax
from jax.experimental import pallas as pl
from jax.experimental.pallas import tpu as pltpu
from jax.experimental.pallas import tpu_sc as plsc

N_NODES = 10000
N_HE = 5000
NNZ = 320000
IN_CH = 128
HIDDEN = 256

NC = 2
NS = 16
NW = NC * NS

NNZ_W = NNZ // NW
CHUNK = 128
N_FULL = NNZ_W // CHUNK
TAIL = NNZ_W - N_FULL * CHUNK

HE_PAD = 5120
ROWS_T = HE_PAD // NS


def _sc_segment_sum(x, node_idx, he_idx, zeros_blk):
    mesh = plsc.VectorSubcoreMesh(
        core_axis_name="c", subcore_axis_name="s", num_cores=NC, num_subcores=NS
    )

    @functools.partial(
        pl.kernel,
        out_type=jax.ShapeDtypeStruct((NC, HE_PAD, IN_CH), jnp.float32),
        mesh=mesh,
        scratch_types=[
            pltpu.VMEM((CHUNK,), jnp.int32),
            pltpu.VMEM((CHUNK,), jnp.int32),
            pltpu.VMEM((CHUNK, IN_CH), jnp.float32),
            pltpu.VMEM((TAIL,), jnp.int32),
            pltpu.VMEM((TAIL,), jnp.int32),
            pltpu.VMEM((TAIL, IN_CH), jnp.float32),
            pltpu.VMEM_SHARED((HE_PAD, IN_CH), jnp.float32),
            pltpu.SemaphoreType.DMA,
        ],
    )
    def seg_sum(x_hbm, ni_hbm, he_hbm, z_hbm, out_hbm,
                ni_v, he_v, rows_v, ni_t, he_t, rows_t, acc, sem):
        cid = lax.axis_index("c")
        sid = lax.axis_index("s")
        wid = cid * NS + sid
        wstart = wid * NNZ_W

        pltpu.sync_copy(z_hbm, acc.at[pl.ds(sid * ROWS_T, ROWS_T)])
        plsc.subcore_barrier()

        def body(i, carry):
            base = wstart + i * CHUNK
            pltpu.sync_copy(ni_hbm.at[pl.ds(base, CHUNK)], ni_v)
            pltpu.sync_copy(he_hbm.at[pl.ds(base, CHUNK)], he_v)
            pltpu.async_copy(x_hbm.at[ni_v], rows_v, sem).wait()
            pltpu.sync_copy(rows_v, acc.at[he_v], add=True)
            return carry

        lax.fori_loop(0, N_FULL, body, 0)

        base = wstart + N_FULL * CHUNK
        pltpu.sync_copy(ni_hbm.at[pl.ds(base, TAIL)], ni_t)
        pltpu.sync_copy(he_hbm.at[pl.ds(base, TAIL)], he_t)
        pltpu.async_copy(x_hbm.at[ni_t], rows_t, sem).wait()
        pltpu.sync_copy(rows_t, acc.at[he_t], add=True)

        plsc.subcore_barrier()
        pltpu.sync_copy(acc.at[pl.ds(sid * ROWS_T, ROWS_T)],
                        out_hbm.at[cid, pl.ds(sid * ROWS_T, ROWS_T)])

    return seg_sum(x, node_idx, he_idx, zeros_blk)


def _tc_body(x_ref, p_ref, w0_ref, b0_ref, w1_ref, b1_ref, x0_ref, x1_ref):
    x0 = jnp.dot(x_ref[...], w0_ref[...], preferred_element_type=jnp.float32)
    x0_ref[...] = jnp.maximum(x0 + b0_ref[...], 0.0)
    x1in = p_ref[0] + p_ref[1]
    x1 = jnp.dot(x1in, w1_ref[...], preferred_element_type=jnp.float32)
    x1_ref[...] = jnp.maximum(x1 + b1_ref[...], 0.0)


def _tc_dense(x, partials, W0, b0, W1, b1):
    grid = 10
    xb = N_NODES // grid
    pb = HE_PAD // grid
    return pl.pallas_call(
        _tc_body,
        grid=(grid,),
        in_specs=[
            pl.BlockSpec((xb, IN_CH), lambda i: (i, 0)),
            pl.BlockSpec((NC, pb, IN_CH), lambda i: (0, i, 0)),
            pl.BlockSpec((IN_CH, HIDDEN), lambda i: (0, 0)),
            pl.BlockSpec((1, HIDDEN), lambda i: (0, 0)),
            pl.BlockSpec((IN_CH, HIDDEN), lambda i: (0, 0)),
            pl.BlockSpec((1, HIDDEN), lambda i: (0, 0)),
        ],
        out_specs=[
            pl.BlockSpec((xb, HIDDEN), lambda i: (i, 0)),
            pl.BlockSpec((pb, HIDDEN), lambda i: (i, 0)),
        ],
        out_shape=[
            jax.ShapeDtypeStruct((N_NODES, HIDDEN), jnp.float32),
            jax.ShapeDtypeStruct((HE_PAD, HIDDEN), jnp.float32),
        ],
    )(x, partials, W0, b0.reshape(1, HIDDEN), W1, b1.reshape(1, HIDDEN))


@jax.jit
def kernel(x, node_idx, he_idx, y, batch_0, W0, b0, W1, b1):
    zeros_blk = jnp.zeros((ROWS_T, IN_CH), jnp.float32)
    partials = _sc_segment_sum(x, node_idx.astype(jnp.int32),
                               he_idx.astype(jnp.int32), zeros_blk)
    x0_out, x1_pad = _tc_dense(x, partials, W0, b0, W1, b1)
    return (x0_out, x1_pad[:N_HE], y, batch_0)


# speedup vs baseline: 10.2540x; 1.5190x over previous
import functools

import jax
import jax.numpy as jnp
from jax import lax
from jax.experimental import pallas as pl
from jax.experimental.pallas import tpu as pltpu
from jax.experimental.pallas import tpu_sc as plsc

N_NODES = 10000
N_HE = 5000
NNZ = 320000
IN_CH = 128
HIDDEN = 256

NC = 2
NS = 16
NW = NC * NS

NNZ_W = NNZ // NW
CHUNK = 128
N_FULL = NNZ_W // CHUNK
TAIL = NNZ_W - N_FULL * CHUNK

HE_PAD = 5120
ROWS_T = HE_PAD // NS


def _sc_segment_sum(x, node_idx, he_idx, zeros_blk):
    mesh = plsc.VectorSubcoreMesh(
        core_axis_name="c", subcore_axis_name="s", num_cores=NC, num_subcores=NS
    )

    @functools.partial(
        pl.kernel,
        out_type=jax.ShapeDtypeStruct((NC, HE_PAD, IN_CH), jnp.float32),
        mesh=mesh,
        scratch_types=[
            pltpu.VMEM((CHUNK,), jnp.int32),
            pltpu.VMEM((CHUNK,), jnp.int32),
            pltpu.VMEM((CHUNK, IN_CH), jnp.float32),
            pltpu.VMEM((CHUNK,), jnp.int32),
            pltpu.VMEM((CHUNK,), jnp.int32),
            pltpu.VMEM((CHUNK, IN_CH), jnp.float32),
            pltpu.VMEM((TAIL,), jnp.int32),
            pltpu.VMEM((TAIL,), jnp.int32),
            pltpu.VMEM((TAIL, IN_CH), jnp.float32),
            pltpu.VMEM_SHARED((HE_PAD, IN_CH), jnp.float32),
            pltpu.SemaphoreType.DMA,
            pltpu.SemaphoreType.DMA,
        ],
    )
    def seg_sum(x_hbm, ni_hbm, he_hbm, z_hbm, out_hbm,
                ni_a, he_a, rows_a, ni_b, he_b, rows_b,
                ni_t, he_t, rows_t, acc, sem_a, sem_b):
        cid = lax.axis_index("c")
        sid = lax.axis_index("s")
        wid = cid * NS + sid
        wstart = wid * NNZ_W

        pltpu.sync_copy(z_hbm, acc.at[pl.ds(sid * ROWS_T, ROWS_T)])
        plsc.subcore_barrier()

        def load_idx(c, ni_v, he_v):
            base = wstart + c * CHUNK
            pltpu.sync_copy(ni_hbm.at[pl.ds(base, CHUNK)], ni_v)
            pltpu.sync_copy(he_hbm.at[pl.ds(base, CHUNK)], he_v)

        load_idx(0, ni_a, he_a)
        pltpu.async_copy(x_hbm.at[ni_a], rows_a, sem_a)

        @pl.loop(0, N_FULL, step=2)
        def _(i):
            load_idx(i + 1, ni_b, he_b)
            pltpu.async_copy(x_hbm.at[ni_b], rows_b, sem_b)
            pltpu.make_async_copy(x_hbm.at[ni_a], rows_a, sem_a).wait()
            pltpu.sync_copy(rows_a, acc.at[he_a], add=True)

            @pl.when(i + 2 < N_FULL)
            def _():
                load_idx(i + 2, ni_a, he_a)
                pltpu.async_copy(x_hbm.at[ni_a], rows_a, sem_a)

            pltpu.make_async_copy(x_hbm.at[ni_b], rows_b, sem_b).wait()
            pltpu.sync_copy(rows_b, acc.at[he_b], add=True)

        base = wstart + N_FULL * CHUNK
        pltpu.sync_copy(ni_hbm.at[pl.ds(base, TAIL)], ni_t)
        pltpu.sync_copy(he_hbm.at[pl.ds(base, TAIL)], he_t)
        pltpu.async_copy(x_hbm.at[ni_t], rows_t, sem_a).wait()
        pltpu.sync_copy(rows_t, acc.at[he_t], add=True)

        plsc.subcore_barrier()
        pltpu.sync_copy(acc.at[pl.ds(sid * ROWS_T, ROWS_T)],
                        out_hbm.at[cid, pl.ds(sid * ROWS_T, ROWS_T)])

    return seg_sum(x, node_idx, he_idx, zeros_blk)


def _tc_body(x_ref, p_ref, w0_ref, b0_ref, w1_ref, b1_ref, x0_ref, x1_ref):
    x0 = jnp.dot(x_ref[...], w0_ref[...], preferred_element_type=jnp.float32)
    x0_ref[...] = jnp.maximum(x0 + b0_ref[...], 0.0)
    x1in = p_ref[0] + p_ref[1]
    x1 = jnp.dot(x1in, w1_ref[...], preferred_element_type=jnp.float32)
    x1_ref[...] = jnp.maximum(x1 + b1_ref[...], 0.0)


def _tc_dense(x, partials, W0, b0, W1, b1):
    grid = 10
    xb = N_NODES // grid
    pb = HE_PAD // grid
    return pl.pallas_call(
        _tc_body,
        grid=(grid,),
        in_specs=[
            pl.BlockSpec((xb, IN_CH), lambda i: (i, 0)),
            pl.BlockSpec((NC, pb, IN_CH), lambda i: (0, i, 0)),
            pl.BlockSpec((IN_CH, HIDDEN), lambda i: (0, 0)),
            pl.BlockSpec((1, HIDDEN), lambda i: (0, 0)),
            pl.BlockSpec((IN_CH, HIDDEN), lambda i: (0, 0)),
            pl.BlockSpec((1, HIDDEN), lambda i: (0, 0)),
        ],
        out_specs=[
            pl.BlockSpec((xb, HIDDEN), lambda i: (i, 0)),
            pl.BlockSpec((pb, HIDDEN), lambda i: (i, 0)),
        ],
        out_shape=[
            jax.ShapeDtypeStruct((N_NODES, HIDDEN), jnp.float32),
            jax.ShapeDtypeStruct((HE_PAD, HIDDEN), jnp.float32),
        ],
    )(x, partials, W0, b0.reshape(1, HIDDEN), W1, b1.reshape(1, HIDDEN))


@jax.jit
def kernel(x, node_idx, he_idx, y, batch_0, W0, b0, W1, b1):
    zeros_blk = jnp.zeros((ROWS_T, IN_CH), jnp.float32)
    partials = _sc_segment_sum(x, node_idx.astype(jnp.int32),
                               he_idx.astype(jnp.int32), zeros_blk)
    x0_out, x1_pad = _tc_dense(x, partials, W0, b0, W1, b1)
    return (x0_out, x1_pad[:N_HE], y, batch_0)


# speedup vs baseline: 12.2301x; 1.1927x over previous
import functools

import jax
import jax.numpy as jnp
from jax import lax
from jax.experimental import pallas as pl
from jax.experimental.pallas import tpu as pltpu
from jax.experimental.pallas import tpu_sc as plsc

N_NODES = 10000
N_HE = 5000
NNZ = 320000
IN_CH = 128
HIDDEN = 256

NC = 2
NS = 16
NW = NC * NS

NNZ_W = NNZ // NW
CHUNK = 128
N_FULL = NNZ_W // CHUNK
TAIL = NNZ_W - N_FULL * CHUNK

HE_PAD = 5120
ROWS_T = HE_PAD // NS

NBUF = 4
N_RING = N_FULL - 2


def _sc_segment_sum(x, node_idx, he_idx, zeros_blk):
    mesh = plsc.VectorSubcoreMesh(
        core_axis_name="c", subcore_axis_name="s", num_cores=NC, num_subcores=NS
    )

    @functools.partial(
        pl.kernel,
        out_type=jax.ShapeDtypeStruct((NC, HE_PAD, IN_CH), jnp.float32),
        mesh=mesh,
        scratch_types=[
            [pltpu.VMEM((CHUNK,), jnp.int32) for _ in range(NBUF)],
            [pltpu.VMEM((CHUNK,), jnp.int32) for _ in range(NBUF)],
            [pltpu.VMEM((CHUNK, IN_CH), jnp.float32) for _ in range(NBUF)],
            pltpu.VMEM((TAIL,), jnp.int32),
            pltpu.VMEM((TAIL,), jnp.int32),
            pltpu.VMEM((TAIL, IN_CH), jnp.float32),
            pltpu.VMEM_SHARED((HE_PAD, IN_CH), jnp.float32),
            [pltpu.SemaphoreType.DMA for _ in range(NBUF)],
            [pltpu.SemaphoreType.DMA for _ in range(NBUF)],
        ],
    )
    def seg_sum(x_hbm, ni_hbm, he_hbm, z_hbm, out_hbm,
                ni, he, rows, ni_t, he_t, rows_t, acc, sem_g, sem_s):
        cid = lax.axis_index("c")
        sid = lax.axis_index("s")
        wid = cid * NS + sid
        wstart = wid * NNZ_W

        pltpu.sync_copy(z_hbm, acc.at[pl.ds(sid * ROWS_T, ROWS_T)])
        plsc.subcore_barrier()

        def start_gather(c, b):
            base = wstart + c * CHUNK
            pltpu.sync_copy(ni_hbm.at[pl.ds(base, CHUNK)], ni[b])
            pltpu.sync_copy(he_hbm.at[pl.ds(base, CHUNK)], he[b])
            pltpu.async_copy(x_hbm.at[ni[b]], rows[b], sem_g[b])

        def wait_gather(b):
            pltpu.make_async_copy(x_hbm.at[ni[b]], rows[b], sem_g[b]).wait()

        def start_scatter(b):
            pltpu.async_copy(rows[b], acc.at[he[b]], sem_s[b], add=True)

        def wait_scatter(b):
            pltpu.make_async_copy(rows[b], acc.at[he[b]], sem_s[b]).wait()

        start_gather(0, 0)
        start_gather(1, 1)

        @pl.loop(0, N_RING, step=NBUF)
        def _(i):
            for b in range(NBUF):
                c = i + b
                wait_gather(b)
                start_scatter(b)

                @pl.when(c >= 2)
                def _():
                    wait_scatter((b + 2) % NBUF)

                @pl.when(c + 2 < N_FULL)
                def _():
                    start_gather(c + 2, (b + 2) % NBUF)

        for b, c in ((0, N_RING), (1, N_RING + 1)):
            wait_gather(b)
            start_scatter(b)
            wait_scatter((b + 2) % NBUF)
        wait_scatter(0)
        wait_scatter(1)

        base = wstart + N_FULL * CHUNK
        pltpu.sync_copy(ni_hbm.at[pl.ds(base, TAIL)], ni_t)
        pltpu.sync_copy(he_hbm.at[pl.ds(base, TAIL)], he_t)
        pltpu.async_copy(x_hbm.at[ni_t], rows_t, sem_g[0]).wait()
        pltpu.sync_copy(rows_t, acc.at[he_t], add=True)

        plsc.subcore_barrier()
        pltpu.sync_copy(acc.at[pl.ds(sid * ROWS_T, ROWS_T)],
                        out_hbm.at[cid, pl.ds(sid * ROWS_T, ROWS_T)])

    return seg_sum(x, node_idx, he_idx, zeros_blk)


def _tc_body(x_ref, p_ref, w0_ref, b0_ref, w1_ref, b1_ref, x0_ref, x1_ref):
    x0 = jnp.dot(x_ref[...], w0_ref[...], preferred_element_type=jnp.float32)
    x0_ref[...] = jnp.maximum(x0 + b0_ref[...], 0.0)
    x1in = p_ref[0] + p_ref[1]
    x1 = jnp.dot(x1in, w1_ref[...], preferred_element_type=jnp.float32)
    x1_ref[...] = jnp.maximum(x1 + b1_ref[...], 0.0)


def _tc_dense(x, partials, W0, b0, W1, b1):
    grid = 10
    xb = N_NODES // grid
    pb = HE_PAD // grid
    return pl.pallas_call(
        _tc_body,
        grid=(grid,),
        in_specs=[
            pl.BlockSpec((xb, IN_CH), lambda i: (i, 0)),
            pl.BlockSpec((NC, pb, IN_CH), lambda i: (0, i, 0)),
            pl.BlockSpec((IN_CH, HIDDEN), lambda i: (0, 0)),
            pl.BlockSpec((1, HIDDEN), lambda i: (0, 0)),
            pl.BlockSpec((IN_CH, HIDDEN), lambda i: (0, 0)),
            pl.BlockSpec((1, HIDDEN), lambda i: (0, 0)),
        ],
        out_specs=[
            pl.BlockSpec((xb, HIDDEN), lambda i: (i, 0)),
            pl.BlockSpec((pb, HIDDEN), lambda i: (i, 0)),
        ],
        out_shape=[
            jax.ShapeDtypeStruct((N_NODES, HIDDEN), jnp.float32),
            jax.ShapeDtypeStruct((HE_PAD, HIDDEN), jnp.float32),
        ],
    )(x, partials, W0, b0.reshape(1, HIDDEN), W1, b1.reshape(1, HIDDEN))


@jax.jit
def kernel(x, node_idx, he_idx, y, batch_0, W0, b0, W1, b1):
    zeros_blk = jnp.zeros((ROWS_T, IN_CH), jnp.float32)
    partials = _sc_segment_sum(x, node_idx.astype(jnp.int32),
                               he_idx.astype(jnp.int32), zeros_blk)
    x0_out, x1_pad = _tc_dense(x, partials, W0, b0, W1, b1)
    return (x0_out, x1_pad[:N_HE], y, batch_0)
